# initial kernel scaffold (unmeasured)
import jax
import jax.numpy as jnp
from jax import lax
from jax.experimental import pallas as pl
from jax.experimental.pallas import tpu as pltpu


def kernel(x, assign, W1, W2):
    t, d = x.shape
    e_loc, _, f = W1.shape
    assign2d = assign.reshape(t, 1)

    def body(x_ref, a_ref, w1_ref, w2_ref, out_ref,
             xsend, xrecv, arecv, ysend, yrecv, send_sems, recv_sems):
        px = lax.axis_index("x")
        py = lax.axis_index("y")
        pz = lax.axis_index("z")
        peer = (1 - px, py, pz)

        xsend[...] = x_ref[...].astype(jnp.bfloat16)

        barrier_sem = pltpu.get_barrier_semaphore()
        pl.semaphore_signal(barrier_sem, inc=1, device_id=peer,
                            device_id_type=pl.DeviceIdType.MESH)
        pl.semaphore_wait(barrier_sem, 1)

        rdma_x = pltpu.make_async_remote_copy(
            src_ref=xsend, dst_ref=xrecv,
            send_sem=send_sems.at[0], recv_sem=recv_sems.at[0],
            device_id=peer, device_id_type=pl.DeviceIdType.MESH)
        rdma_x.start()
        rdma_a = pltpu.make_async_remote_copy(
            src_ref=a_ref, dst_ref=arecv,
            send_sem=send_sems.at[1], recv_sem=recv_sems.at[1],
            device_id=peer, device_id_type=pl.DeviceIdType.MESH)
        rdma_a.start()

        def moe_local(xb, ab):
            acc = jnp.zeros((t, d), jnp.float32)
            for le in range(e_loc):
                eg = px * e_loc + le
                mask = (ab == eg).astype(jnp.bfloat16)
                xm = xb * mask
                w1b = w1_ref[le].astype(jnp.bfloat16)
                w2b = w2_ref[le].astype(jnp.bfloat16)
                h = jnp.dot(xm, w1b, preferred_element_type=jnp.float32)
                hb = jnp.maximum(h, 0.0).astype(jnp.bfloat16)
                acc = acc + jnp.dot(hb, w2b,
                                    preferred_element_type=jnp.float32)
            return acc

        out_ref[...] = moe_local(xsend[...], a_ref[...])

        rdma_x.wait()
        rdma_a.wait()

        ysend[...] = moe_local(xrecv[...], arecv[...]).astype(jnp.bfloat16)
        rdma_y = pltpu.make_async_remote_copy(
            src_ref=ysend, dst_ref=yrecv,
            send_sem=send_sems.at[2], recv_sem=recv_sems.at[2],
            device_id=peer, device_id_type=pl.DeviceIdType.MESH)
        rdma_y.start()
        rdma_y.wait()

        out_ref[...] = out_ref[...] + yrecv[...].astype(jnp.float32)

    return pl.pallas_call(
        body,
        out_shape=jax.ShapeDtypeStruct((t, d), jnp.float32),
        in_specs=[
            pl.BlockSpec(memory_space=pltpu.VMEM),
            pl.BlockSpec(memory_space=pltpu.VMEM),
            pl.BlockSpec(memory_space=pltpu.VMEM),
            pl.BlockSpec(memory_space=pltpu.VMEM),
        ],
        out_specs=pl.BlockSpec(memory_space=pltpu.VMEM),
        scratch_shapes=[
            pltpu.VMEM((t, d), jnp.bfloat16),
            pltpu.VMEM((t, d), jnp.bfloat16),
            pltpu.VMEM((t, 1), jnp.int32),
            pltpu.VMEM((t, d), jnp.bfloat16),
            pltpu.VMEM((t, d), jnp.bfloat16),
            pltpu.SemaphoreType.DMA((3,)),
            pltpu.SemaphoreType.DMA((3,)),
        ],
        compiler_params=pltpu.CompilerParams(collective_id=0),
    )(x, assign2d, W1, W2)


# baseline (device time: 95839 ns/iter reference)
import jax
import jax.numpy as jnp
from jax import lax
from jax.experimental import pallas as pl
from jax.experimental.pallas import tpu as pltpu


def kernel(x, assign, W1, W2):
    t, d = x.shape
    e_loc, _, f = W1.shape
    assign2d = assign.reshape(t, 1)

    def body(x_ref, a_ref, w1_ref, w2_ref, out_ref,
             xsend, xrecv, arecv, ysend, yrecv, send_sems, recv_sems):
        px = lax.axis_index("x")
        py = lax.axis_index("y")
        pz = lax.axis_index("z")
        peer = (1 - px, py, pz)

        xsend[...] = x_ref[...].astype(jnp.bfloat16)

        barrier_sem = pltpu.get_barrier_semaphore()
        pl.semaphore_signal(barrier_sem, inc=1, device_id=peer,
                            device_id_type=pl.DeviceIdType.MESH)
        pl.semaphore_wait(barrier_sem, 1)

        rdma_x = pltpu.make_async_remote_copy(
            src_ref=xsend, dst_ref=xrecv,
            send_sem=send_sems.at[0], recv_sem=recv_sems.at[0],
            device_id=peer, device_id_type=pl.DeviceIdType.MESH)
        rdma_x.start()
        rdma_a = pltpu.make_async_remote_copy(
            src_ref=a_ref, dst_ref=arecv,
            send_sem=send_sems.at[1], recv_sem=recv_sems.at[1],
            device_id=peer, device_id_type=pl.DeviceIdType.MESH)
        rdma_a.start()

        def moe_local(xb, ab):
            acc = jnp.zeros((t, d), jnp.float32)
            for le in range(e_loc):
                eg = px * e_loc + le
                mask = (ab == eg).astype(jnp.bfloat16)
                xm = xb * mask
                w1b = w1_ref[le].astype(jnp.bfloat16)
                w2b = w2_ref[le].astype(jnp.bfloat16)
                h = jnp.dot(xm, w1b, preferred_element_type=jnp.float32)
                hb = jnp.maximum(h, 0.0).astype(jnp.bfloat16)
                acc = acc + jnp.dot(hb, w2b,
                                    preferred_element_type=jnp.float32)
            return acc

        out_ref[...] = moe_local(xsend[...], a_ref[...])

        rdma_x.wait()
        rdma_a.wait()

        ysend[...] = moe_local(xrecv[...], arecv[...]).astype(jnp.bfloat16)
        rdma_y = pltpu.make_async_remote_copy(
            src_ref=ysend, dst_ref=yrecv,
            send_sem=send_sems.at[2], recv_sem=recv_sems.at[2],
            device_id=peer, device_id_type=pl.DeviceIdType.MESH)
        rdma_y.start()
        rdma_y.wait()

        out_ref[...] = out_ref[...] + yrecv[...].astype(jnp.float32)

    return pl.pallas_call(
        body,
        out_shape=jax.ShapeDtypeStruct((t, d), jnp.float32),
        in_specs=[
            pl.BlockSpec(memory_space=pltpu.VMEM),
            pl.BlockSpec(memory_space=pltpu.VMEM),
            pl.BlockSpec(memory_space=pltpu.VMEM),
            pl.BlockSpec(memory_space=pltpu.VMEM),
        ],
        out_specs=pl.BlockSpec(memory_space=pltpu.VMEM),
        scratch_shapes=[
            pltpu.VMEM((t, d), jnp.bfloat16),
            pltpu.VMEM((t, d), jnp.bfloat16),
            pltpu.VMEM((t, 1), jnp.int32),
            pltpu.VMEM((t, d), jnp.bfloat16),
            pltpu.VMEM((t, d), jnp.bfloat16),
            pltpu.SemaphoreType.DMA((3,)),
            pltpu.SemaphoreType.DMA((3,)),
        ],
        compiler_params=pltpu.CompilerParams(
            collective_id=0,
            vmem_limit_bytes=100 * 1024 * 1024,
        ),
    )(x, assign2d, W1, W2)
